# baseline (speedup 1.0000x reference)
import functools

import jax
import jax.numpy as jnp
from jax import lax
from jax.experimental import pallas as pl
from jax.experimental.pallas import tpu as pltpu
from jax.experimental.pallas import tpu_sc as plsc

NC = 2
NS = 16
NW = NC * NS
CHUNK = 100
GB = 2


def kernel(input_ids, attention_mask, embed_table):
    B, S = input_ids.shape
    V, D = embed_table.shape
    per_b = B // NW
    per_w = per_b * S
    n_chunks = per_w // CHUNK
    GCH = GB * S // CHUNK
    n_groups = per_b // GB
    ids = input_ids.reshape(NW, n_chunks, CHUNK).astype(jnp.int32)

    mesh = plsc.VectorSubcoreMesh(core_axis_name="c", subcore_axis_name="s")

    @functools.partial(
        pl.kernel,
        out_type=jax.ShapeDtypeStruct((B, S, D), jnp.float32),
        mesh=mesh,
        scratch_types=[
            pltpu.VMEM((n_chunks, CHUNK), jnp.int32),
            pltpu.VMEM((2, GB, S, D), jnp.float32),
            pltpu.VMEM_SHARED((V, D), jnp.float32),
            pltpu.SemaphoreType.DMA((2,)),
            pltpu.SemaphoreType.DMA((2,)),
        ],
        compiler_params=pltpu.CompilerParams(use_tc_tiling_on_sc=False),
    )
    def emb(ids_hbm, table_hbm, out_hbm, idx_v, rows_v, table_sh, gsem, wsem):
        wid = lax.axis_index("s") * NC + lax.axis_index("c")
        bbase = wid * per_b
        sid = lax.axis_index("s")

        @pl.when(sid == 0)
        def _():
            pltpu.sync_copy(table_hbm, table_sh)

        pltpu.sync_copy(ids_hbm.at[wid], idx_v)
        plsc.subcore_barrier()

        def gather_parts(g, b):
            parts = []
            for k in range(GCH):
                t = k * CHUNK
                parts.append(
                    (
                        table_sh.at[idx_v.at[g * GCH + k]],
                        rows_v.at[b, t // S, pl.ds(t % S, CHUNK)],
                    )
                )
            return parts

        def fire_gathers(g, b):
            for src, dst in gather_parts(g, b):
                pltpu.async_copy(src, dst, gsem.at[b])

        def wait_gathers(g, b):
            for src, dst in gather_parts(g, b):
                pltpu.make_async_copy(src, dst, gsem.at[b]).wait()

        def fire_write(g, b):
            pltpu.async_copy(
                rows_v.at[b], out_hbm.at[pl.ds(bbase + g * GB, GB)], wsem.at[b]
            )

        def wait_write(b):
            pltpu.make_async_copy(
                rows_v.at[b], out_hbm.at[pl.ds(bbase, GB)], wsem.at[b]
            ).wait()

        fire_gathers(0, 0)

        def body(gg, carry):
            for b in range(2):
                g = gg * 2 + b
                wait_gathers(g, b)
                fire_write(g, b)
                ob = 1 - b

                @pl.when(g + 1 < n_groups)
                def _():
                    @pl.when(g >= 1)
                    def _():
                        wait_write(ob)

                    fire_gathers(g + 1, ob)

            return carry

        lax.fori_loop(0, n_groups // 2, body, 0)
        wait_write(0)
        wait_write(1)

    return emb(ids, embed_table) + 0.0


# speedup vs baseline: 1.6495x; 1.6495x over previous
import functools

import jax
import jax.numpy as jnp
from jax import lax
from jax.experimental import pallas as pl
from jax.experimental.pallas import tpu as pltpu
from jax.experimental.pallas import tpu_sc as plsc

NC = 2
NS = 16
NW = NC * NS
CHUNK = 100
DP = 128


def kernel(input_ids, attention_mask, embed_table):
    B, S = input_ids.shape
    V, D = embed_table.shape
    per_b = B // NW
    per_w = per_b * S
    n_chunks = per_w // CHUNK
    GCH = S // CHUNK
    n_groups = per_b
    ids = input_ids.reshape(NW, n_chunks, CHUNK).astype(jnp.int32)
    table_p = jnp.pad(embed_table, ((0, 0), (0, DP - D)))

    mesh = plsc.VectorSubcoreMesh(core_axis_name="c", subcore_axis_name="s")

    @functools.partial(
        pl.kernel,
        out_type=jax.ShapeDtypeStruct((B, S, DP), jnp.float32),
        mesh=mesh,
        scratch_types=[
            pltpu.VMEM((n_chunks, CHUNK), jnp.int32),
            pltpu.VMEM((2, 1, S, DP), jnp.float32),
            pltpu.VMEM_SHARED((V, DP), jnp.float32),
            pltpu.SemaphoreType.DMA((2,)),
            pltpu.SemaphoreType.DMA((2,)),
        ],
    )
    def emb(ids_hbm, table_hbm, out_hbm, idx_v, rows_v, table_sh, gsem, wsem):
        wid = lax.axis_index("s") * NC + lax.axis_index("c")
        bbase = wid * per_b
        sid = lax.axis_index("s")

        @pl.when(sid == 0)
        def _():
            pltpu.sync_copy(table_hbm, table_sh)

        pltpu.sync_copy(ids_hbm.at[wid], idx_v)
        plsc.subcore_barrier()

        def gather_parts(g, b):
            return [
                (
                    table_sh.at[idx_v.at[g * GCH + k]],
                    rows_v.at[b, 0, pl.ds(k * CHUNK, CHUNK)],
                )
                for k in range(GCH)
            ]

        def fire_gathers(g, b):
            for src, dst in gather_parts(g, b):
                pltpu.async_copy(src, dst, gsem.at[b])

        def wait_gathers(g, b):
            for src, dst in gather_parts(g, b):
                pltpu.make_async_copy(src, dst, gsem.at[b]).wait()

        def fire_write(g, b):
            pltpu.async_copy(
                rows_v.at[b], out_hbm.at[pl.ds(bbase + g, 1)], wsem.at[b]
            )

        def wait_write(b):
            pltpu.make_async_copy(
                rows_v.at[b], out_hbm.at[pl.ds(bbase, 1)], wsem.at[b]
            ).wait()

        fire_gathers(0, 0)

        def body(gg, carry):
            for b in range(2):
                g = gg * 2 + b
                wait_gathers(g, b)
                fire_write(g, b)
                ob = 1 - b

                @pl.when(g + 1 < n_groups)
                def _():
                    @pl.when(g >= 1)
                    def _():
                        wait_write(ob)

                    fire_gathers(g + 1, ob)

            return carry

        lax.fori_loop(0, n_groups // 2, body, 0)
        wait_write(0)
        wait_write(1)

    out = emb(ids, table_p)
    return out[:, :, :D]
